# baseline (speedup 1.0000x reference)
import jax
import jax.numpy as jnp
from jax.experimental import pallas as pl
from jax.experimental.pallas import tpu as pltpu

_B = 64; _K = 256; _NN = 64; _OBS = 8; _PRED = 12; _INS = 2
_E = 64; _H = 4; _FF = 128
_DH = _E // _H
_PK = 100
_NK = 20
_DIN = _OBS * _INS
_DMODE = _PRED * 2

_PARAM_ORDER = (
    'W_emb', 'b_emb',
    'enc_Wq', 'enc_bq', 'enc_Wk', 'enc_bk', 'enc_Wv', 'enc_bv',
    'enc_Wo', 'enc_bo', 'enc_ln1_g', 'enc_ln1_b',
    'enc_W1', 'enc_b1', 'enc_W2', 'enc_b2', 'enc_ln2_g', 'enc_ln2_b',
    'dec_Wq', 'dec_bq', 'dec_Wk', 'dec_bk', 'dec_Wv', 'dec_bv',
    'dec_Wo', 'dec_bo', 'dec_ln1_g', 'dec_ln1_b',
    'dec_W1', 'dec_b1', 'dec_W2', 'dec_b2', 'dec_ln2_g', 'dec_ln2_b',
    'W_cls', 'b_cls', 'W_cls2', 'b_cls2', 'W_nei', 'b_nei',
    'W_reg', 'b_reg',
)


_PREC = jax.lax.Precision.HIGHEST


def _dot(a, b):
    return jax.lax.dot_general(a, b, (((1,), (0,)), ((), ())),
                               preferred_element_type=jnp.float32)


def _dot_x(a, b):
    return jax.lax.dot_general(a, b, (((1,), (0,)), ((), ())),
                               preferred_element_type=jnp.float32,
                               precision=_PREC)


def _dot_t(a, b):
    return jax.lax.dot_general(a, b, (((1,), (1,)), ((), ())),
                               preferred_element_type=jnp.float32)


def _col_to_row(v, n):
    i0 = jax.lax.broadcasted_iota(jnp.int32, (n, n), 0)
    i1 = jax.lax.broadcasted_iota(jnp.int32, (n, n), 1)
    ident = jnp.where(i0 == i1, 1.0, 0.0)
    return jax.lax.dot_general(v, ident, (((0,), (0,)), ((), ())),
                               preferred_element_type=jnp.float32,
                               precision=_PREC)


def _row_to_col(v, n):
    i0 = jax.lax.broadcasted_iota(jnp.int32, (n, n), 0)
    i1 = jax.lax.broadcasted_iota(jnp.int32, (n, n), 1)
    ident = jnp.where(i0 == i1, 1.0, 0.0)
    return jax.lax.dot_general(ident, v, (((1,), (1,)), ((), ())),
                               preferred_element_type=jnp.float32,
                               precision=_PREC)


def _xla_row_sum(x):
    w = x.shape[-1]
    if w > 128:
        x = x[:, :128] + x[:, 128:]
        w = 128
    if w % 8:
        pad = 8 - w % 8
        x = jnp.concatenate(
            [x, jnp.zeros((x.shape[0], pad), x.dtype)], axis=1)
        w += pad
    acc = x[:, 0:8]
    for i in range(1, w // 8):
        acc = acc + x[:, 8 * i:8 * i + 8]
    acc = acc[:, 0:4] + acc[:, 4:8]
    acc = acc[:, 0:2] + acc[:, 2:4]
    return acc[:, 0:1] + acc[:, 1:2]


def _ln(x, g, b):
    n = float(x.shape[-1])
    m = _xla_row_sum(x) / n
    d = x - m
    v = _xla_row_sum(d * d) / n
    return d / jnp.sqrt(v + 1e-5) * g + b


def _softmax_rows(s):
    m = jnp.max(s, axis=-1, keepdims=True)
    e = jnp.exp(s - m)
    return e / _xla_row_sum(e)


def _softmax_dot(s, v):
    m = jnp.max(s, axis=-1, keepdims=True)
    e = jnp.exp(s - m)
    return _dot(e, v) / _xla_row_sum(e)


def _rank_row(s_col, s_row, n):
    ia = jax.lax.broadcasted_iota(jnp.int32, (n, n), 0)
    ib = jax.lax.broadcasted_iota(jnp.int32, (n, n), 1)
    gt = s_col > s_row
    eq = (s_col == s_row) & (ia < ib)
    d = jnp.where(gt | eq, 1.0, 0.0)
    return jnp.sum(d, axis=0, keepdims=True)


def _select_rows(rank_row, feats, k, n):
    m_iota = jax.lax.broadcasted_iota(jnp.int32, (k, n), 0)
    rank_i = jnp.broadcast_to(rank_row, (k, n)).astype(jnp.int32)
    sel = jnp.where(rank_i == m_iota, 1.0, 0.0)
    return _dot_x(sel, feats)


def _attn(q, k, v, heads, mask_row=None):
    outs = []
    for h in range(heads):
        sl = slice(h * _DH, (h + 1) * _DH)
        s = _dot_t(q[:, sl], k[:, sl]) * 0.25
        if mask_row is not None:
            s = jnp.where(mask_row > 0, s, -1e9)
        outs.append(_softmax_dot(s, v[:, sl]))
    return jnp.concatenate(outs, axis=1)


def _body(ped_ref, neis_ref, modes_ref, maskrow_ref, *refs):
    out_pred_ref, out_sn_ref = refs[-2], refs[-1]
    p = dict(zip(_PARAM_ORDER, refs[:-2]))

    inp = jnp.concatenate(
        [jnp.broadcast_to(ped_ref[0], (_K, _DIN)), modes_ref[...]], axis=1)
    x = _dot(inp, p['W_emb'][...]) + p['b_emb'][...]

    q = _dot(x, p['enc_Wq'][...]) + p['enc_bq'][...]
    k = _dot(x, p['enc_Wk'][...]) + p['enc_bk'][...]
    v = _dot(x, p['enc_Wv'][...]) + p['enc_bv'][...]
    a = _dot(_attn(q, k, v, _H), p['enc_Wo'][...]) + p['enc_bo'][...]
    x = _ln(x + a, p['enc_ln1_g'][...], p['enc_ln1_b'][...])
    h = jnp.maximum(_dot(x, p['enc_W1'][...]) + p['enc_b1'][...], 0.0)
    h = _dot(h, p['enc_W2'][...]) + p['enc_b2'][...]
    pf = _ln(x + h, p['enc_ln2_g'][...], p['enc_ln2_b'][...])

    sc_col = _dot(pf, p['W_cls'][...]) + p['b_cls'][...]
    sc_row = _col_to_row(sc_col, _K)
    rank = _rank_row(sc_col, sc_row, _K)
    topf = _select_rows(rank, pf, _PK, _K)

    ne = _dot(neis_ref[0], p['W_nei'][...]) + p['b_nei'][...]
    q2 = _dot(topf, p['dec_Wq'][...]) + p['dec_bq'][...]
    k2 = _dot(ne, p['dec_Wk'][...]) + p['dec_bk'][...]
    v2 = _dot(ne, p['dec_Wv'][...]) + p['dec_bv'][...]
    a2 = _attn(q2, k2, v2, _H, mask_row=maskrow_ref[0])
    a2 = _dot(a2, p['dec_Wo'][...]) + p['dec_bo'][...]
    x2 = _ln(topf + a2, p['dec_ln1_g'][...], p['dec_ln1_b'][...])
    h2 = jnp.maximum(_dot(x2, p['dec_W1'][...]) + p['dec_b1'][...], 0.0)
    h2 = _dot(h2, p['dec_W2'][...]) + p['dec_b2'][...]
    intf = _ln(x2 + h2, p['dec_ln2_g'][...], p['dec_ln2_b'][...])

    lg = _dot(intf, p['W_cls2'][...]) + p['b_cls2'][...]
    lg_row = _col_to_row(lg, _PK)
    mx = jnp.max(lg_row, axis=-1, keepdims=True)
    e = jnp.exp(lg_row - mx)
    sn_row = e / _xla_row_sum(e)
    sn_col = _row_to_col(sn_row, _PK)
    out_sn_ref[0] = sn_row

    rank2 = _rank_row(sn_col, sn_row, _PK)
    top2 = _select_rows(rank2, intf, _NK, _PK)
    out_pred_ref[0] = _dot(top2, p['W_reg'][...]) + p['b_reg'][...]


def kernel(ped_obs, neis_obs, motion_modes, mask, closest_mode_indices,
           num_k, ped_num_k, params):
    bb = ped_obs.shape[0]
    ped = ped_obs.reshape(bb, 1, _DIN).astype(jnp.float32)
    neis = neis_obs.reshape(bb, _NN, _DIN).astype(jnp.float32)
    modes = motion_modes.reshape(_K, _DMODE).astype(jnp.float32)
    maskrow = mask[:, 0:1, :].astype(jnp.float32)

    pargs = []
    for name in _PARAM_ORDER:
        w = params[name]
        if w.ndim == 1:
            w = w.reshape(1, -1)
        pargs.append(w.astype(jnp.float32))

    in_specs = [
        pl.BlockSpec((1, 1, _DIN), lambda b: (b, 0, 0)),
        pl.BlockSpec((1, _NN, _DIN), lambda b: (b, 0, 0)),
        pl.BlockSpec((_K, _DMODE), lambda b: (0, 0)),
        pl.BlockSpec((1, 1, _NN), lambda b: (b, 0, 0)),
    ] + [pl.BlockSpec(w.shape, lambda b, nd=w.ndim: (0,) * nd) for w in pargs]

    out_specs = (
        pl.BlockSpec((1, _NK, 2 * _PRED), lambda b: (b, 0, 0)),
        pl.BlockSpec((1, 1, _PK), lambda b: (b, 0, 0)),
    )
    out_shape = (
        jax.ShapeDtypeStruct((bb, _NK, 2 * _PRED), jnp.float32),
        jax.ShapeDtypeStruct((bb, 1, _PK), jnp.float32),
    )

    pred, sn = pl.pallas_call(
        _body,
        grid=(bb,),
        in_specs=in_specs,
        out_specs=out_specs,
        out_shape=out_shape,
        compiler_params=pltpu.CompilerParams(
            dimension_semantics=("parallel",)),
    )(ped, neis, modes, maskrow, *pargs)
    return (pred, sn.reshape(bb, _PK))
